# Initial kernel scaffold; baseline (speedup 1.0000x reference)
import jax
import jax.numpy as jnp
from jax.experimental import pallas as pl


def kernel(rating_matrix_batch, rating_matrix_batch2, gram_matrix, W1, b1, W2, b2, items, epsilon):
    raise NotImplementedError("write your pallas kernel here")


# speedup vs baseline: 1.6734x; 1.6734x over previous
import functools

import jax
import jax.numpy as jnp
from jax import lax
from jax.experimental import pallas as pl
from jax.experimental.pallas import tpu as pltpu
from jax.experimental.pallas import tpu_sc as plsc

NUM_ITEMS = 8192
BATCH = 1024
ENC_H = 600
Z_DIM = 200
TAU = 0.2
EPS = 1e-12

_NC = 2
_NS = 16
_NW = _NC * _NS
_ROWS_PER_W = BATCH // _NW
_CH = 8
_NCHUNK = _ROWS_PER_W // _CH

_BM = 128
_NBLK = BATCH // _BM


def _sc_gather(gram, idx2d):
    mesh = plsc.VectorSubcoreMesh(core_axis_name="c", subcore_axis_name="s")

    @functools.partial(
        pl.kernel,
        mesh=mesh,
        out_type=jax.ShapeDtypeStruct((BATCH, NUM_ITEMS), jnp.float32),
        scratch_types=[
            pltpu.VMEM((_NCHUNK, _CH), jnp.int32),
            pltpu.VMEM((_CH, NUM_ITEMS), jnp.float32),
            pltpu.SemaphoreType.DMA,
        ],
    )
    def gather_kernel(table_hbm, idx_hbm, out_hbm, idx_v, buf, sem):
        wid = lax.axis_index("s") * _NC + lax.axis_index("c")
        base = wid * _ROWS_PER_W
        pltpu.sync_copy(idx_hbm.at[pl.ds(wid * _NCHUNK, _NCHUNK)], idx_v)
        for c in range(_NCHUNK):
            pltpu.async_copy(table_hbm.at[idx_v.at[c]], buf, sem).wait()
            pltpu.sync_copy(buf, out_hbm.at[pl.ds(base + c * _CH, _CH)])

    return gather_kernel(gram, idx2d)


def _tc_body(gath_ref, rate_ref, w1_ref, b1_ref, w2_ref, b2_ref, itemsT_ref,
             eps_ref, z_ref, logits_ref, klrow_ref, itn_ref, acc_ref):
    i = pl.program_id(0)

    @pl.when(i == 0)
    def _():
        it = itemsT_ref[...]
        s = jnp.sqrt(jnp.sum(it * it, axis=0, keepdims=True))
        itn_ref[...] = (it / jnp.maximum(s, EPS)).astype(jnp.bfloat16)

    v = jnp.where(rate_ref[...] > 0, gath_ref[...], 0.0)
    s1 = jnp.maximum(jnp.sum(jnp.abs(v), axis=1, keepdims=True), EPS)
    l2 = jnp.sqrt(jnp.sum(v * v, axis=1, keepdims=True)) / s1
    binp = v * (1.0 / (s1 * jnp.maximum(l2, EPS)))

    h = jnp.tanh(
        jnp.dot(binp.astype(jnp.bfloat16), w1_ref[...],
                preferred_element_type=jnp.float32) + b1_ref[...])
    x2 = jnp.dot(h, w2_ref[...], preferred_element_type=jnp.float32) + b2_ref[...]
    mean = x2[:, :Z_DIM]
    logvar = x2[:, Z_DIM:]
    std = jnp.exp(0.5 * logvar)
    z = mean + eps_ref[...] * std
    z_ref[...] = z

    zn = z / jnp.maximum(jnp.sqrt(jnp.sum(z * z, axis=1, keepdims=True)), EPS)
    logits_ref[...] = jnp.dot(zn.astype(jnp.bfloat16), itn_ref[...],
                              preferred_element_type=jnp.float32) * (1.0 / TAU)

    var = std * std
    klb = jnp.sum(mean * mean + var - 1.0 - logvar)
    prev = jnp.where(i == 0, 0.0, acc_ref[0, 0])
    total = prev + klb
    acc_ref[0, 0] = total
    klrow_ref[...] = jnp.full((1, 1, 128), total * (0.5 / BATCH), jnp.float32)


def _tc_encoder(gathered, rating, W1b, b1, W2, b2, itemsT, epsilon):
    return pl.pallas_call(
        _tc_body,
        grid=(_NBLK,),
        in_specs=[
            pl.BlockSpec((_BM, NUM_ITEMS), lambda i: (i, 0)),
            pl.BlockSpec((_BM, NUM_ITEMS), lambda i: (i, 0)),
            pl.BlockSpec((NUM_ITEMS, ENC_H), lambda i: (0, 0)),
            pl.BlockSpec((1, ENC_H), lambda i: (0, 0)),
            pl.BlockSpec((ENC_H, 2 * Z_DIM), lambda i: (0, 0)),
            pl.BlockSpec((1, 2 * Z_DIM), lambda i: (0, 0)),
            pl.BlockSpec((Z_DIM, NUM_ITEMS), lambda i: (0, 0)),
            pl.BlockSpec((_BM, Z_DIM), lambda i: (i, 0)),
        ],
        out_specs=[
            pl.BlockSpec((_BM, Z_DIM), lambda i: (i, 0)),
            pl.BlockSpec((_BM, NUM_ITEMS), lambda i: (i, 0)),
            pl.BlockSpec((1, 1, 128), lambda i: (i, 0, 0)),
        ],
        out_shape=[
            jax.ShapeDtypeStruct((BATCH, Z_DIM), jnp.float32),
            jax.ShapeDtypeStruct((BATCH, NUM_ITEMS), jnp.float32),
            jax.ShapeDtypeStruct((_NBLK, 1, 128), jnp.float32),
        ],
        scratch_shapes=[
            pltpu.VMEM((Z_DIM, NUM_ITEMS), jnp.bfloat16),
            pltpu.SMEM((1, 1), jnp.float32),
        ],
    )(gathered, rating, W1b, b1, W2, b2, itemsT, epsilon)


def kernel(rating_matrix_batch, rating_matrix_batch2, gram_matrix, W1, b1, W2,
           b2, items, epsilon):
    idx2d = rating_matrix_batch2.astype(jnp.int32).reshape(BATCH // _CH, _CH)
    gathered = _sc_gather(gram_matrix, idx2d)
    z, logits, klrows = _tc_encoder(
        gathered, rating_matrix_batch, W1.astype(jnp.bfloat16),
        b1.reshape(1, ENC_H), W2, b2.reshape(1, 2 * Z_DIM), items.T, epsilon)
    kl = klrows[_NBLK - 1, 0, 0]
    return z, logits, kl
